# baseline (speedup 1.0000x reference)
import math

import jax
import jax.numpy as jnp
from jax import lax
from jax.experimental import pallas as pl
from jax.experimental.pallas import tpu as pltpu
from jax.experimental.pallas import tpu_sc as plsc

N = 10000
E = 320000
D = 128
H = 128
NCONV = 2
LAMDA = 0.5
ALPHA = 0.1
NR = 9000
NRP = 9088
K = 128

NCORES = 2
NSUB = 16
NTILES = NCORES * NSUB
EPT = E // NTILES
CH2 = 80
CPT2 = EPT // CH2
CH4 = 100
CPT4 = EPT // CH4
RPS = N // NSUB

_vec_mesh = plsc.VectorSubcoreMesh(core_axis_name="c", subcore_axis_name="s")
_sc_params = pltpu.CompilerParams(use_tc_tiling_on_sc=False,
                                  needs_layout_passes=False)


def _dense_pre_body(x_ref, wh_ref, bh_ref, w0_ref, b0_ref, w1_ref, b1_ref,
                    av_ref, h_ref, al_ref, ar_ref, armax_ref):
    h = jnp.maximum(
        jnp.dot(x_ref[...], wh_ref[...], preferred_element_type=jnp.float32)
        + bh_ref[...], 0.0)
    h_ref[...] = h
    hl = jnp.dot(h, w0_ref[...], preferred_element_type=jnp.float32) + b0_ref[...]
    hr = jnp.dot(h, w1_ref[...], preferred_element_type=jnp.float32) + b1_ref[...]
    av = av_ref[...]

    def lrelu(v):
        return jnp.where(v > 0, v, 0.2 * v)

    al = jnp.sum(lrelu(hl) * av, axis=1, keepdims=True)
    ar = jnp.sum(lrelu(hr) * av, axis=1, keepdims=True)
    al_ref[...] = al
    ar_ref[...] = ar
    armax_ref[...] = jnp.reshape(jnp.max(ar), (1, 1))


def _dense_pre(x, W_hidden, b_hidden, Wsa0, bsa0, Wsa1, bsa1, a_vec):
    return pl.pallas_call(
        _dense_pre_body,
        out_shape=(
            jax.ShapeDtypeStruct((N, H), jnp.float32),
            jax.ShapeDtypeStruct((N, 1), jnp.float32),
            jax.ShapeDtypeStruct((N, 1), jnp.float32),
            jax.ShapeDtypeStruct((1, 1), jnp.float32),
        ),
    )(x, W_hidden, b_hidden.reshape(1, H), Wsa0, bsa0.reshape(1, H),
      Wsa1, bsa1.reshape(1, H), a_vec.reshape(1, H))


def _edge_stats_body(src_hbm, dst_hbm, al_hbm, ar_hbm, armax_hbm,
                     rs2_out, emax_out,
                     altab, artab, qtab, srcv, dstv, sidx, vals, maxv,
                     armax_v, rs2_sh):
    c = lax.axis_index("c")
    s = lax.axis_index("s")
    wid = c * NSUB + s

    pltpu.sync_copy(al_hbm, altab)
    pltpu.sync_copy(ar_hbm, artab)
    pltpu.sync_copy(armax_hbm, armax_v)
    pltpu.sync_copy(src_hbm.at[pl.ds(wid * CPT2, CPT2)], srcv)
    pltpu.sync_copy(dst_hbm.at[pl.ds(wid * CPT2, CPT2)], dstv)

    @pl.when(s == 0)
    def _():
        @pl.loop(0, N, step=16)
        def _(i):
            qtab[pl.ds(i, 16)] = jnp.zeros((16,), jnp.float32)
        pltpu.sync_copy(qtab, rs2_sh)

    am = armax_v[...]

    @pl.loop(0, N, step=16)
    def _(i):
        qtab[pl.ds(i, 16)] = jnp.exp(artab[pl.ds(i, 16)] - am)

    maxv[...] = jnp.full((16,), -jnp.inf, jnp.float32)

    plsc.subcore_barrier()

    @pl.loop(0, CPT2)
    def _(ch):
        @pl.loop(0, CH2, step=16)
        def _(j):
            sv = srcv[ch, pl.ds(j, 16)]
            dv = dstv[ch, pl.ds(j, 16)]
            m = plsc.load_gather(altab, [sv]) + plsc.load_gather(artab, [dv])
            maxv[...] = jnp.maximum(maxv[...], m)
            vals[pl.ds(j, 16)] = plsc.load_gather(qtab, [dv])
            sidx[pl.ds(j, 16)] = sv
        pltpu.sync_copy(vals, rs2_sh.at[sidx], add=True)

    plsc.subcore_barrier()

    @pl.when(s == 0)
    def _():
        pltpu.sync_copy(rs2_sh, rs2_out.at[c])
    pltpu.sync_copy(maxv, emax_out.at[wid])


def _edge_stats(src2, dst2, al, ar, armax16):
    kfn = pl.kernel(
        _edge_stats_body,
        out_type=(
            jax.ShapeDtypeStruct((NCORES, N), jnp.float32),
            jax.ShapeDtypeStruct((NTILES, 16), jnp.float32),
        ),
        mesh=_vec_mesh,
        scratch_types=[
            pltpu.VMEM((N,), jnp.float32),
            pltpu.VMEM((N,), jnp.float32),
            pltpu.VMEM((N,), jnp.float32),
            pltpu.VMEM((CPT2, CH2), jnp.int32),
            pltpu.VMEM((CPT2, CH2), jnp.int32),
            pltpu.VMEM((CH2,), jnp.int32),
            pltpu.VMEM((CH2,), jnp.float32),
            pltpu.VMEM((16,), jnp.float32),
            pltpu.VMEM((16,), jnp.float32),
            pltpu.VMEM_SHARED((N,), jnp.float32),
        ],
        compiler_params=_sc_params,
    )
    return kfn(src2, dst2, al, ar, armax16)


def _prep_body(rs2p_ref, emaxp_ref, al_ref, ar_ref, armax_ref, h_ref,
               q_ref, p_ref, feat_ref):
    emax = jnp.max(emaxp_ref[...])
    armax = armax_ref[...]
    rs2 = rs2p_ref[0] + rs2p_ref[1]
    t = jnp.exp(al_ref[...] - emax + armax)
    p = t / (t * rs2 + 1e-16)
    q = jnp.exp(ar_ref[...] - armax)
    q_ref[...] = q
    p_ref[...] = p
    feat_ref[...] = p * h_ref[...]


def _prep(rs2p, emaxp, al, ar, armax, h):
    return pl.pallas_call(
        _prep_body,
        out_shape=(
            jax.ShapeDtypeStruct((N, 1), jnp.float32),
            jax.ShapeDtypeStruct((N, 1), jnp.float32),
            jax.ShapeDtypeStruct((N, H), jnp.float32),
        ),
    )(rs2p, emaxp, al, ar, armax, h)


def _spmm_body(feat_hbm, src_hbm, dst_hbm, out_hbm, sidx, didx, rows0, rows1,
               acc, sem0, sem1):
    c = lax.axis_index("c")
    s = lax.axis_index("s")
    wid = c * NSUB + s

    pltpu.sync_copy(src_hbm.at[pl.ds(wid * CPT4, CPT4)], sidx)
    pltpu.sync_copy(dst_hbm.at[pl.ds(wid * CPT4, CPT4)], didx)

    @pl.loop(0, CH4)
    def _(r):
        @pl.loop(0, D, step=16)
        def _(k):
            rows0[r, pl.ds(k, 16)] = jnp.zeros((16,), jnp.float32)

    @pl.loop(0, RPS // CH4)
    def _(j):
        pltpu.sync_copy(rows0, acc.at[pl.ds(s * RPS + j * CH4, CH4)])

    pltpu.sync_copy(rows0, acc.at[pl.ds(s * RPS + RPS - CH4, CH4)])

    plsc.subcore_barrier()

    pltpu.async_copy(feat_hbm.at[sidx.at[0]], rows0, sem0)

    @pl.loop(0, CPT4, step=2)
    def _(ch):
        pltpu.async_copy(feat_hbm.at[sidx.at[ch + 1]], rows1, sem1)
        pltpu.make_async_copy(feat_hbm.at[sidx.at[ch]], rows0, sem0).wait()
        pltpu.sync_copy(rows0, acc.at[didx.at[ch]], add=True)

        @pl.when(ch + 2 < CPT4)
        def _():
            pltpu.async_copy(feat_hbm.at[sidx.at[ch + 2]], rows0, sem0)

        pltpu.make_async_copy(feat_hbm.at[sidx.at[ch + 1]], rows1, sem1).wait()
        pltpu.sync_copy(rows1, acc.at[didx.at[ch + 1]], add=True)

    plsc.subcore_barrier()

    pltpu.sync_copy(acc.at[pl.ds(s * RPS, RPS)],
                    out_hbm.at[c, pl.ds(s * RPS, RPS)])


def _spmm(feat, src4, dst4):
    kfn = pl.kernel(
        _spmm_body,
        out_type=jax.ShapeDtypeStruct((NCORES, N, D), jnp.float32),
        mesh=_vec_mesh,
        scratch_types=[
            pltpu.VMEM((CPT4, CH4), jnp.int32),
            pltpu.VMEM((CPT4, CH4), jnp.int32),
            pltpu.VMEM((CH4, D), jnp.float32),
            pltpu.VMEM((CH4, D), jnp.float32),
            pltpu.VMEM_SHARED((N, D), jnp.float32),
            pltpu.SemaphoreType.DMA,
            pltpu.SemaphoreType.DMA,
        ],
        compiler_params=_sc_params,
    )
    return kfn(feat, src4, dst4)


def _make_layer_body(theta):
    def body(parts_ref, h0_ref, q_ref, p_ref, w_ref, feat_ref):
        hi = q_ref[...] * (parts_ref[0] + parts_ref[1])
        support = (1.0 - ALPHA) * hi + ALPHA * h0_ref[...]
        out = theta * jnp.dot(support, w_ref[...],
                              preferred_element_type=jnp.float32) \
            + (1.0 - theta) * support
        feat_ref[...] = p_ref[...] * jnp.maximum(out, 0.0)
    return body


def _layer_update(theta, parts, h0, q, p, W_init):
    return pl.pallas_call(
        _make_layer_body(theta),
        out_shape=jax.ShapeDtypeStruct((N, H), jnp.float32),
    )(parts, h0, q, p, W_init)


def _topk_body(scores_ref, rni_ref, sel_ref, scr):
    scr[...] = scores_ref[...]
    rows_i = lax.broadcasted_iota(jnp.int32, (NRP // 128, 128), 0)
    cols_i = lax.broadcasted_iota(jnp.int32, (NRP // 128, 128), 1)
    flat = rows_i * 128 + cols_i

    def tk_body(t, carry):
        sv = scr[...]
        m = jnp.max(sv)
        sel = jnp.min(jnp.where(sv == m, flat, jnp.int32(1 << 30)))
        sel_ref[t] = rni_ref[sel]
        scr[...] = jnp.where(flat == sel, -jnp.inf, sv)
        return carry

    lax.fori_loop(0, K, tk_body, 0)


def _topk(scores_pad, rni):
    return pl.pallas_call(
        _topk_body,
        out_shape=jax.ShapeDtypeStruct((K,), jnp.int32),
        in_specs=[
            pl.BlockSpec(memory_space=pltpu.VMEM),
            pl.BlockSpec(memory_space=pltpu.SMEM),
        ],
        out_specs=pl.BlockSpec(memory_space=pltpu.SMEM),
        scratch_shapes=[
            pltpu.VMEM((NRP // 128, 128), jnp.float32),
        ],
    )(scores_pad, rni)


def _make_pool_gru_body(theta):
    def body(parts_ref, h0_ref, q_ref, sel_ref, wi_ref, wih_ref, whh_ref,
             bih_ref, bhh_ref, out_ref, layer, xp, gi):
        hi = q_ref[...] * (parts_ref[0] + parts_ref[1])
        support = (1.0 - ALPHA) * hi + ALPHA * h0_ref[...]
        out = theta * jnp.dot(support, wi_ref[...],
                              preferred_element_type=jnp.float32) \
            + (1.0 - theta) * support
        layer[...] = jnp.maximum(out, 0.0)

        def gather_body(t, carry):
            nid = sel_ref[t]
            xp[pl.ds(t, 1), :] = layer[pl.ds(nid, 1), :]
            return carry

        lax.fori_loop(0, K, gather_body, 0)

        gi[...] = lax.dot_general(xp[...], wih_ref[...],
                                  (((1,), (1,)), ((), ())),
                                  preferred_element_type=jnp.float32) \
            + bih_ref[...]

        def gru_body(t, hv):
            gh = lax.dot_general(hv, whh_ref[...], (((1,), (1,)), ((), ())),
                                 preferred_element_type=jnp.float32) \
                + bhh_ref[...]
            git = gi[pl.ds(t, 1), :]
            r = jax.nn.sigmoid(git[:, 0:H] + gh[:, 0:H])
            z = jax.nn.sigmoid(git[:, H:2 * H] + gh[:, H:2 * H])
            n = jnp.tanh(git[:, 2 * H:3 * H] + r * gh[:, 2 * H:3 * H])
            hn = (1.0 - z) * n + z * hv
            out_ref[pl.ds(t, 1), :] = hn
            return hn

        lax.fori_loop(0, K, gru_body, jnp.zeros((1, H), jnp.float32))

    return body


def _pool_gru(theta, parts, h0, q, sel, W_init, W_ih, W_hh, b_ih, b_hh):
    return pl.pallas_call(
        _make_pool_gru_body(theta),
        out_shape=jax.ShapeDtypeStruct((K, H), jnp.float32),
        in_specs=[
            pl.BlockSpec(memory_space=pltpu.VMEM),
            pl.BlockSpec(memory_space=pltpu.VMEM),
            pl.BlockSpec(memory_space=pltpu.VMEM),
            pl.BlockSpec(memory_space=pltpu.SMEM),
            pl.BlockSpec(memory_space=pltpu.VMEM),
            pl.BlockSpec(memory_space=pltpu.VMEM),
            pl.BlockSpec(memory_space=pltpu.VMEM),
            pl.BlockSpec(memory_space=pltpu.VMEM),
            pl.BlockSpec(memory_space=pltpu.VMEM),
        ],
        scratch_shapes=[
            pltpu.VMEM((N, H), jnp.float32),
            pltpu.VMEM((K, H), jnp.float32),
            pltpu.VMEM((K, 3 * H), jnp.float32),
        ],
    )(parts, h0, q, sel, W_init, W_ih, W_hh, b_ih, b_hh)


def kernel(x, edge_index, remain_nodes_index, added_nodes_index, node_id,
           node_scores, W_hidden, b_hidden, Wsa0, bsa0, Wsa1, bsa1, a_vec,
           W_init, W_ih, W_hh, b_ih, b_hh):
    src = edge_index[0]
    dst = edge_index[1]
    src2 = src.reshape(E // CH2, CH2)
    dst2 = dst.reshape(E // CH2, CH2)
    src4 = src.reshape(E // CH4, CH4)
    dst4 = dst.reshape(E // CH4, CH4)

    h, al, ar, armax = _dense_pre(x, W_hidden, b_hidden, Wsa0, bsa0,
                                  Wsa1, bsa1, a_vec)
    armax16 = jnp.broadcast_to(armax.reshape(1), (16,))
    rs2p, emaxp = _edge_stats(src2, dst2, al.reshape(N), ar.reshape(N),
                              armax16)

    scores_pad = jnp.pad(node_scores, (0, NRP - NR),
                         constant_values=-jnp.inf).reshape(NRP // 128, 128)
    sel = _topk(scores_pad, remain_nodes_index)

    q, p, feat = _prep(rs2p.reshape(NCORES, N, 1), emaxp, al, ar, armax, h)

    theta1 = math.log(LAMDA / 1 + 1.0)
    parts = _spmm(feat, src4, dst4)
    feat = _layer_update(theta1, parts, h, q, p, W_init)

    theta2 = math.log(LAMDA / 2 + 1.0)
    parts = _spmm(feat, src4, dst4)
    return _pool_gru(theta2, parts, h, q, sel, W_init, W_ih, W_hh,
                     b_ih.reshape(1, 3 * H), b_hh.reshape(1, 3 * H))


# speedup vs baseline: 1.0952x; 1.0952x over previous
import math

import jax
import jax.numpy as jnp
from jax import lax
from jax.experimental import pallas as pl
from jax.experimental.pallas import tpu as pltpu
from jax.experimental.pallas import tpu_sc as plsc

N = 10000
E = 320000
D = 128
H = 128
NCONV = 2
LAMDA = 0.5
ALPHA = 0.1
NR = 9000
NRP = 9088
K = 128

NCORES = 2
NSUB = 16
NTILES = NCORES * NSUB
EPT = E // NTILES
CH2 = 80
CPT2 = EPT // CH2
CH4 = 50
NBUF = 4
CPT4 = EPT // CH4
RPS = N // NSUB

_vec_mesh = plsc.VectorSubcoreMesh(core_axis_name="c", subcore_axis_name="s")
_sc_params = pltpu.CompilerParams(use_tc_tiling_on_sc=False,
                                  needs_layout_passes=False)


def _dense_pre_body(x_ref, wh_ref, bh_ref, w0_ref, b0_ref, w1_ref, b1_ref,
                    av_ref, h_ref, al_ref, ar_ref, armax_ref):
    h = jnp.maximum(
        jnp.dot(x_ref[...], wh_ref[...], preferred_element_type=jnp.float32)
        + bh_ref[...], 0.0)
    h_ref[...] = h
    hl = jnp.dot(h, w0_ref[...], preferred_element_type=jnp.float32) + b0_ref[...]
    hr = jnp.dot(h, w1_ref[...], preferred_element_type=jnp.float32) + b1_ref[...]
    av = av_ref[...]

    def lrelu(v):
        return jnp.where(v > 0, v, 0.2 * v)

    al = jnp.sum(lrelu(hl) * av, axis=1, keepdims=True)
    ar = jnp.sum(lrelu(hr) * av, axis=1, keepdims=True)
    al_ref[...] = al
    ar_ref[...] = ar
    armax_ref[...] = jnp.reshape(jnp.max(ar), (1, 1))


def _dense_pre(x, W_hidden, b_hidden, Wsa0, bsa0, Wsa1, bsa1, a_vec):
    return pl.pallas_call(
        _dense_pre_body,
        out_shape=(
            jax.ShapeDtypeStruct((N, H), jnp.float32),
            jax.ShapeDtypeStruct((N, 1), jnp.float32),
            jax.ShapeDtypeStruct((N, 1), jnp.float32),
            jax.ShapeDtypeStruct((1, 1), jnp.float32),
        ),
    )(x, W_hidden, b_hidden.reshape(1, H), Wsa0, bsa0.reshape(1, H),
      Wsa1, bsa1.reshape(1, H), a_vec.reshape(1, H))


def _edge_stats_body(src_hbm, dst_hbm, al_hbm, ar_hbm, armax_hbm,
                     rs2_out, emax_out,
                     altab, artab, qtab, srcv, dstv, sidx, vals, maxv,
                     armax_v, rs2_sh):
    c = lax.axis_index("c")
    s = lax.axis_index("s")
    wid = c * NSUB + s

    pltpu.sync_copy(al_hbm, altab)
    pltpu.sync_copy(ar_hbm, artab)
    pltpu.sync_copy(armax_hbm, armax_v)
    pltpu.sync_copy(src_hbm.at[pl.ds(wid * CPT2, CPT2)], srcv)
    pltpu.sync_copy(dst_hbm.at[pl.ds(wid * CPT2, CPT2)], dstv)

    @pl.when(s == 0)
    def _():
        @pl.loop(0, N, step=16)
        def _(i):
            qtab[pl.ds(i, 16)] = jnp.zeros((16,), jnp.float32)
        pltpu.sync_copy(qtab, rs2_sh)

    am = armax_v[...]

    @pl.loop(0, N, step=16)
    def _(i):
        qtab[pl.ds(i, 16)] = jnp.exp(artab[pl.ds(i, 16)] - am)

    maxv[...] = jnp.full((16,), -jnp.inf, jnp.float32)

    plsc.subcore_barrier()

    @pl.loop(0, CPT2)
    def _(ch):
        @pl.loop(0, CH2, step=16)
        def _(j):
            sv = srcv[ch, pl.ds(j, 16)]
            dv = dstv[ch, pl.ds(j, 16)]
            m = plsc.load_gather(altab, [sv]) + plsc.load_gather(artab, [dv])
            maxv[...] = jnp.maximum(maxv[...], m)
            vals[pl.ds(j, 16)] = plsc.load_gather(qtab, [dv])
            sidx[pl.ds(j, 16)] = sv
        pltpu.sync_copy(vals, rs2_sh.at[sidx], add=True)

    plsc.subcore_barrier()

    @pl.when(s == 0)
    def _():
        pltpu.sync_copy(rs2_sh, rs2_out.at[c])
    pltpu.sync_copy(maxv, emax_out.at[wid])


def _edge_stats(src2, dst2, al, ar, armax16):
    kfn = pl.kernel(
        _edge_stats_body,
        out_type=(
            jax.ShapeDtypeStruct((NCORES, N), jnp.float32),
            jax.ShapeDtypeStruct((NTILES, 16), jnp.float32),
        ),
        mesh=_vec_mesh,
        scratch_types=[
            pltpu.VMEM((N,), jnp.float32),
            pltpu.VMEM((N,), jnp.float32),
            pltpu.VMEM((N,), jnp.float32),
            pltpu.VMEM((CPT2, CH2), jnp.int32),
            pltpu.VMEM((CPT2, CH2), jnp.int32),
            pltpu.VMEM((CH2,), jnp.int32),
            pltpu.VMEM((CH2,), jnp.float32),
            pltpu.VMEM((16,), jnp.float32),
            pltpu.VMEM((16,), jnp.float32),
            pltpu.VMEM_SHARED((N,), jnp.float32),
        ],
        compiler_params=_sc_params,
    )
    return kfn(src2, dst2, al, ar, armax16)


def _prep_body(rs2p_ref, emaxp_ref, al_ref, ar_ref, armax_ref, h_ref,
               q_ref, p_ref, feat_ref):
    emax = jnp.max(emaxp_ref[...])
    armax = armax_ref[...]
    rs2 = rs2p_ref[0] + rs2p_ref[1]
    t = jnp.exp(al_ref[...] - emax + armax)
    p = t / (t * rs2 + 1e-16)
    q = jnp.exp(ar_ref[...] - armax)
    q_ref[...] = q
    p_ref[...] = p
    feat_ref[...] = p * h_ref[...]


def _prep(rs2p, emaxp, al, ar, armax, h):
    return pl.pallas_call(
        _prep_body,
        out_shape=(
            jax.ShapeDtypeStruct((N, 1), jnp.float32),
            jax.ShapeDtypeStruct((N, 1), jnp.float32),
            jax.ShapeDtypeStruct((N, H), jnp.float32),
        ),
    )(rs2p, emaxp, al, ar, armax, h)


def _spmm_body(feat_hbm, src_hbm, dst_hbm, out_hbm, sidx, didx,
               rows0, rows1, rows2, rows3, acc, sem0, sem1, sem2, sem3):
    c = lax.axis_index("c")
    s = lax.axis_index("s")
    wid = c * NSUB + s
    bufs = (rows0, rows1, rows2, rows3)
    sems = (sem0, sem1, sem2, sem3)

    pltpu.sync_copy(src_hbm.at[pl.ds(wid * CPT4, CPT4)], sidx)
    pltpu.sync_copy(dst_hbm.at[pl.ds(wid * CPT4, CPT4)], didx)

    @pl.loop(0, CH4)
    def _(r):
        @pl.loop(0, D, step=16)
        def _(k):
            rows0[r, pl.ds(k, 16)] = jnp.zeros((16,), jnp.float32)

    @pl.loop(0, RPS // CH4)
    def _(j):
        pltpu.sync_copy(rows0, acc.at[pl.ds(s * RPS + j * CH4, CH4)])

    pltpu.sync_copy(rows0, acc.at[pl.ds(s * RPS + RPS - CH4, CH4)])

    plsc.subcore_barrier()

    for b in range(NBUF):
        pltpu.async_copy(feat_hbm.at[sidx.at[b]], bufs[b], sems[b])

    @pl.loop(0, CPT4, step=NBUF)
    def _(ch):
        for b in range(NBUF):
            pltpu.make_async_copy(feat_hbm.at[sidx.at[ch + b]],
                                  bufs[b], sems[b]).wait()
            pltpu.sync_copy(bufs[b], acc.at[didx.at[ch + b]], add=True)

            @pl.when(ch + b + NBUF < CPT4)
            def _(b=b):
                pltpu.async_copy(feat_hbm.at[sidx.at[ch + b + NBUF]],
                                 bufs[b], sems[b])

    plsc.subcore_barrier()

    pltpu.sync_copy(acc.at[pl.ds(s * RPS, RPS)],
                    out_hbm.at[c, pl.ds(s * RPS, RPS)])


def _spmm(feat, src4, dst4):
    kfn = pl.kernel(
        _spmm_body,
        out_type=jax.ShapeDtypeStruct((NCORES, N, D), jnp.float32),
        mesh=_vec_mesh,
        scratch_types=[
            pltpu.VMEM((CPT4, CH4), jnp.int32),
            pltpu.VMEM((CPT4, CH4), jnp.int32),
            pltpu.VMEM((CH4, D), jnp.float32),
            pltpu.VMEM((CH4, D), jnp.float32),
            pltpu.VMEM((CH4, D), jnp.float32),
            pltpu.VMEM((CH4, D), jnp.float32),
            pltpu.VMEM_SHARED((N, D), jnp.float32),
            pltpu.SemaphoreType.DMA,
            pltpu.SemaphoreType.DMA,
            pltpu.SemaphoreType.DMA,
            pltpu.SemaphoreType.DMA,
        ],
        compiler_params=_sc_params,
    )
    return kfn(feat, src4, dst4)


def _make_layer_body(theta):
    def body(parts_ref, h0_ref, q_ref, p_ref, w_ref, feat_ref):
        hi = q_ref[...] * (parts_ref[0] + parts_ref[1])
        support = (1.0 - ALPHA) * hi + ALPHA * h0_ref[...]
        out = theta * jnp.dot(support, w_ref[...],
                              preferred_element_type=jnp.float32) \
            + (1.0 - theta) * support
        feat_ref[...] = p_ref[...] * jnp.maximum(out, 0.0)
    return body


def _layer_update(theta, parts, h0, q, p, W_init):
    return pl.pallas_call(
        _make_layer_body(theta),
        out_shape=jax.ShapeDtypeStruct((N, H), jnp.float32),
    )(parts, h0, q, p, W_init)


def _topk_body(scores_ref, rni_ref, sel_ref, scr):
    scr[...] = scores_ref[...]
    rows_i = lax.broadcasted_iota(jnp.int32, (NRP // 128, 128), 0)
    cols_i = lax.broadcasted_iota(jnp.int32, (NRP // 128, 128), 1)
    flat = rows_i * 128 + cols_i

    def tk_body(t, carry):
        sv = scr[...]
        m = jnp.max(sv)
        sel = jnp.min(jnp.where(sv == m, flat, jnp.int32(1 << 30)))
        sel_ref[t] = rni_ref[sel]
        scr[...] = jnp.where(flat == sel, -jnp.inf, sv)
        return carry

    lax.fori_loop(0, K, tk_body, 0)


def _topk(scores_pad, rni):
    return pl.pallas_call(
        _topk_body,
        out_shape=jax.ShapeDtypeStruct((K,), jnp.int32),
        in_specs=[
            pl.BlockSpec(memory_space=pltpu.VMEM),
            pl.BlockSpec(memory_space=pltpu.SMEM),
        ],
        out_specs=pl.BlockSpec(memory_space=pltpu.SMEM),
        scratch_shapes=[
            pltpu.VMEM((NRP // 128, 128), jnp.float32),
        ],
    )(scores_pad, rni)


def _make_pool_gru_body(theta):
    def body(parts_ref, h0_ref, q_ref, sel_ref, wi_ref, wih_ref, whh_ref,
             bih_ref, bhh_ref, out_ref, layer, xp, gi):
        hi = q_ref[...] * (parts_ref[0] + parts_ref[1])
        support = (1.0 - ALPHA) * hi + ALPHA * h0_ref[...]
        out = theta * jnp.dot(support, wi_ref[...],
                              preferred_element_type=jnp.float32) \
            + (1.0 - theta) * support
        layer[...] = jnp.maximum(out, 0.0)

        def gather_body(t, carry):
            nid = sel_ref[t]
            xp[pl.ds(t, 1), :] = layer[pl.ds(nid, 1), :]
            return carry

        lax.fori_loop(0, K, gather_body, 0)

        gi[...] = lax.dot_general(xp[...], wih_ref[...],
                                  (((1,), (1,)), ((), ())),
                                  preferred_element_type=jnp.float32) \
            + bih_ref[...]

        def gru_body(t, hv):
            gh = lax.dot_general(hv, whh_ref[...], (((1,), (1,)), ((), ())),
                                 preferred_element_type=jnp.float32) \
                + bhh_ref[...]
            git = gi[pl.ds(t, 1), :]
            r = jax.nn.sigmoid(git[:, 0:H] + gh[:, 0:H])
            z = jax.nn.sigmoid(git[:, H:2 * H] + gh[:, H:2 * H])
            n = jnp.tanh(git[:, 2 * H:3 * H] + r * gh[:, 2 * H:3 * H])
            hn = (1.0 - z) * n + z * hv
            out_ref[pl.ds(t, 1), :] = hn
            return hn

        lax.fori_loop(0, K, gru_body, jnp.zeros((1, H), jnp.float32))

    return body


def _pool_gru(theta, parts, h0, q, sel, W_init, W_ih, W_hh, b_ih, b_hh):
    return pl.pallas_call(
        _make_pool_gru_body(theta),
        out_shape=jax.ShapeDtypeStruct((K, H), jnp.float32),
        in_specs=[
            pl.BlockSpec(memory_space=pltpu.VMEM),
            pl.BlockSpec(memory_space=pltpu.VMEM),
            pl.BlockSpec(memory_space=pltpu.VMEM),
            pl.BlockSpec(memory_space=pltpu.SMEM),
            pl.BlockSpec(memory_space=pltpu.VMEM),
            pl.BlockSpec(memory_space=pltpu.VMEM),
            pl.BlockSpec(memory_space=pltpu.VMEM),
            pl.BlockSpec(memory_space=pltpu.VMEM),
            pl.BlockSpec(memory_space=pltpu.VMEM),
        ],
        scratch_shapes=[
            pltpu.VMEM((N, H), jnp.float32),
            pltpu.VMEM((K, H), jnp.float32),
            pltpu.VMEM((K, 3 * H), jnp.float32),
        ],
    )(parts, h0, q, sel, W_init, W_ih, W_hh, b_ih, b_hh)


def kernel(x, edge_index, remain_nodes_index, added_nodes_index, node_id,
           node_scores, W_hidden, b_hidden, Wsa0, bsa0, Wsa1, bsa1, a_vec,
           W_init, W_ih, W_hh, b_ih, b_hh):
    src = edge_index[0]
    dst = edge_index[1]
    src2 = src.reshape(E // CH2, CH2)
    dst2 = dst.reshape(E // CH2, CH2)
    src4 = src.reshape(E // CH4, CH4)
    dst4 = dst.reshape(E // CH4, CH4)

    h, al, ar, armax = _dense_pre(x, W_hidden, b_hidden, Wsa0, bsa0,
                                  Wsa1, bsa1, a_vec)
    armax16 = jnp.broadcast_to(armax.reshape(1), (16,))
    rs2p, emaxp = _edge_stats(src2, dst2, al.reshape(N), ar.reshape(N),
                              armax16)

    scores_pad = jnp.pad(node_scores, (0, NRP - NR),
                         constant_values=-jnp.inf).reshape(NRP // 128, 128)
    sel = _topk(scores_pad, remain_nodes_index)

    q, p, feat = _prep(rs2p.reshape(NCORES, N, 1), emaxp, al, ar, armax, h)

    theta1 = math.log(LAMDA / 1 + 1.0)
    parts = _spmm(feat, src4, dst4)
    feat = _layer_update(theta1, parts, h, q, p, W_init)

    theta2 = math.log(LAMDA / 2 + 1.0)
    parts = _spmm(feat, src4, dst4)
    return _pool_gru(theta2, parts, h, q, sel, W_init, W_ih, W_hh,
                     b_ih.reshape(1, 3 * H), b_hh.reshape(1, 3 * H))


# speedup vs baseline: 1.1073x; 1.0111x over previous
import math

import jax
import jax.numpy as jnp
from jax import lax
from jax.experimental import pallas as pl
from jax.experimental.pallas import tpu as pltpu
from jax.experimental.pallas import tpu_sc as plsc

N = 10000
E = 320000
D = 128
H = 128
NCONV = 2
LAMDA = 0.5
ALPHA = 0.1
NR = 9000
NRP = 9088
K = 128

NCORES = 2
NSUB = 16
NTILES = NCORES * NSUB
EPT = E // NTILES
CH2 = 80
CPT2 = EPT // CH2
CH4 = 40
NBUF = 5
CPT4 = EPT // CH4
RPS = N // NSUB

_vec_mesh = plsc.VectorSubcoreMesh(core_axis_name="c", subcore_axis_name="s")
_sc_params = pltpu.CompilerParams(use_tc_tiling_on_sc=False,
                                  needs_layout_passes=False)


def _dense_pre_body(x_ref, wh_ref, bh_ref, w0_ref, b0_ref, w1_ref, b1_ref,
                    av_ref, h_ref, al_ref, ar_ref, armax_ref):
    h = jnp.maximum(
        jnp.dot(x_ref[...], wh_ref[...], preferred_element_type=jnp.float32)
        + bh_ref[...], 0.0)
    h_ref[...] = h
    hl = jnp.dot(h, w0_ref[...], preferred_element_type=jnp.float32) + b0_ref[...]
    hr = jnp.dot(h, w1_ref[...], preferred_element_type=jnp.float32) + b1_ref[...]
    av = av_ref[...]

    def lrelu(v):
        return jnp.where(v > 0, v, 0.2 * v)

    al = jnp.sum(lrelu(hl) * av, axis=1, keepdims=True)
    ar = jnp.sum(lrelu(hr) * av, axis=1, keepdims=True)
    al_ref[...] = al
    ar_ref[...] = ar
    armax_ref[...] = jnp.reshape(jnp.max(ar), (1, 1))


def _dense_pre(x, W_hidden, b_hidden, Wsa0, bsa0, Wsa1, bsa1, a_vec):
    return pl.pallas_call(
        _dense_pre_body,
        out_shape=(
            jax.ShapeDtypeStruct((N, H), jnp.float32),
            jax.ShapeDtypeStruct((N, 1), jnp.float32),
            jax.ShapeDtypeStruct((N, 1), jnp.float32),
            jax.ShapeDtypeStruct((1, 1), jnp.float32),
        ),
    )(x, W_hidden, b_hidden.reshape(1, H), Wsa0, bsa0.reshape(1, H),
      Wsa1, bsa1.reshape(1, H), a_vec.reshape(1, H))


def _edge_stats_body(src_hbm, dst_hbm, al_hbm, ar_hbm, armax_hbm,
                     rs2_out, emax_out,
                     altab, artab, qtab, srcv, dstv, sidx, vals, maxv,
                     armax_v, rs2_sh):
    c = lax.axis_index("c")
    s = lax.axis_index("s")
    wid = c * NSUB + s

    pltpu.sync_copy(al_hbm, altab)
    pltpu.sync_copy(ar_hbm, artab)
    pltpu.sync_copy(armax_hbm, armax_v)
    pltpu.sync_copy(src_hbm.at[pl.ds(wid * CPT2, CPT2)], srcv)
    pltpu.sync_copy(dst_hbm.at[pl.ds(wid * CPT2, CPT2)], dstv)

    @pl.when(s == 0)
    def _():
        @pl.loop(0, N, step=16)
        def _(i):
            qtab[pl.ds(i, 16)] = jnp.zeros((16,), jnp.float32)
        pltpu.sync_copy(qtab, rs2_sh)

    am = armax_v[...]

    @pl.loop(0, N, step=16)
    def _(i):
        qtab[pl.ds(i, 16)] = jnp.exp(artab[pl.ds(i, 16)] - am)

    maxv[...] = jnp.full((16,), -jnp.inf, jnp.float32)

    plsc.subcore_barrier()

    @pl.loop(0, CPT2)
    def _(ch):
        @pl.loop(0, CH2, step=16)
        def _(j):
            sv = srcv[ch, pl.ds(j, 16)]
            dv = dstv[ch, pl.ds(j, 16)]
            m = plsc.load_gather(altab, [sv]) + plsc.load_gather(artab, [dv])
            maxv[...] = jnp.maximum(maxv[...], m)
            vals[pl.ds(j, 16)] = plsc.load_gather(qtab, [dv])
            sidx[pl.ds(j, 16)] = sv
        pltpu.sync_copy(vals, rs2_sh.at[sidx], add=True)

    plsc.subcore_barrier()

    @pl.when(s == 0)
    def _():
        pltpu.sync_copy(rs2_sh, rs2_out.at[c])
    pltpu.sync_copy(maxv, emax_out.at[wid])


def _edge_stats(src2, dst2, al, ar, armax16):
    kfn = pl.kernel(
        _edge_stats_body,
        out_type=(
            jax.ShapeDtypeStruct((NCORES, N), jnp.float32),
            jax.ShapeDtypeStruct((NTILES, 16), jnp.float32),
        ),
        mesh=_vec_mesh,
        scratch_types=[
            pltpu.VMEM((N,), jnp.float32),
            pltpu.VMEM((N,), jnp.float32),
            pltpu.VMEM((N,), jnp.float32),
            pltpu.VMEM((CPT2, CH2), jnp.int32),
            pltpu.VMEM((CPT2, CH2), jnp.int32),
            pltpu.VMEM((CH2,), jnp.int32),
            pltpu.VMEM((CH2,), jnp.float32),
            pltpu.VMEM((16,), jnp.float32),
            pltpu.VMEM((16,), jnp.float32),
            pltpu.VMEM_SHARED((N,), jnp.float32),
        ],
        compiler_params=_sc_params,
    )
    return kfn(src2, dst2, al, ar, armax16)


def _prep_body(rs2p_ref, emaxp_ref, al_ref, ar_ref, armax_ref, h_ref,
               q_ref, p_ref, feat_ref):
    emax = jnp.max(emaxp_ref[...])
    armax = armax_ref[...]
    rs2 = rs2p_ref[0] + rs2p_ref[1]
    t = jnp.exp(al_ref[...] - emax + armax)
    p = t / (t * rs2 + 1e-16)
    q = jnp.exp(ar_ref[...] - armax)
    q_ref[...] = q
    p_ref[...] = p
    feat_ref[...] = p * h_ref[...]


def _prep(rs2p, emaxp, al, ar, armax, h):
    return pl.pallas_call(
        _prep_body,
        out_shape=(
            jax.ShapeDtypeStruct((N, 1), jnp.float32),
            jax.ShapeDtypeStruct((N, 1), jnp.float32),
            jax.ShapeDtypeStruct((N, H), jnp.float32),
        ),
    )(rs2p, emaxp, al, ar, armax, h)


def _spmm_body(feat_hbm, src_hbm, dst_hbm, out_hbm, sidx, didx,
               rows0, rows1, rows2, rows3, rows4, acc,
               sem0, sem1, sem2, sem3, sem4):
    c = lax.axis_index("c")
    s = lax.axis_index("s")
    wid = c * NSUB + s
    bufs = (rows0, rows1, rows2, rows3, rows4)
    sems = (sem0, sem1, sem2, sem3, sem4)

    pltpu.sync_copy(src_hbm.at[pl.ds(wid * CPT4, CPT4)], sidx)
    pltpu.sync_copy(dst_hbm.at[pl.ds(wid * CPT4, CPT4)], didx)

    @pl.loop(0, CH4)
    def _(r):
        @pl.loop(0, D, step=16)
        def _(k):
            rows0[r, pl.ds(k, 16)] = jnp.zeros((16,), jnp.float32)

    @pl.loop(0, RPS // CH4)
    def _(j):
        pltpu.sync_copy(rows0, acc.at[pl.ds(s * RPS + j * CH4, CH4)])

    pltpu.sync_copy(rows0, acc.at[pl.ds(s * RPS + RPS - CH4, CH4)])

    plsc.subcore_barrier()

    for b in range(NBUF):
        pltpu.async_copy(feat_hbm.at[sidx.at[b]], bufs[b], sems[b])

    @pl.loop(0, CPT4, step=NBUF)
    def _(ch):
        for b in range(NBUF):
            pltpu.make_async_copy(feat_hbm.at[sidx.at[ch + b]],
                                  bufs[b], sems[b]).wait()
            pltpu.sync_copy(bufs[b], acc.at[didx.at[ch + b]], add=True)

            @pl.when(ch + b + NBUF < CPT4)
            def _(b=b):
                pltpu.async_copy(feat_hbm.at[sidx.at[ch + b + NBUF]],
                                 bufs[b], sems[b])

    plsc.subcore_barrier()

    pltpu.sync_copy(acc.at[pl.ds(s * RPS, RPS)],
                    out_hbm.at[c, pl.ds(s * RPS, RPS)])


def _spmm(feat, src4, dst4):
    kfn = pl.kernel(
        _spmm_body,
        out_type=jax.ShapeDtypeStruct((NCORES, N, D), jnp.float32),
        mesh=_vec_mesh,
        scratch_types=[
            pltpu.VMEM((CPT4, CH4), jnp.int32),
            pltpu.VMEM((CPT4, CH4), jnp.int32),
            pltpu.VMEM((CH4, D), jnp.float32),
            pltpu.VMEM((CH4, D), jnp.float32),
            pltpu.VMEM((CH4, D), jnp.float32),
            pltpu.VMEM((CH4, D), jnp.float32),
            pltpu.VMEM((CH4, D), jnp.float32),
            pltpu.VMEM_SHARED((N, D), jnp.float32),
            pltpu.SemaphoreType.DMA,
            pltpu.SemaphoreType.DMA,
            pltpu.SemaphoreType.DMA,
            pltpu.SemaphoreType.DMA,
            pltpu.SemaphoreType.DMA,
        ],
        compiler_params=_sc_params,
    )
    return kfn(feat, src4, dst4)


def _make_layer_body(theta):
    def body(parts_ref, h0_ref, q_ref, p_ref, w_ref, feat_ref):
        hi = q_ref[...] * (parts_ref[0] + parts_ref[1])
        support = (1.0 - ALPHA) * hi + ALPHA * h0_ref[...]
        out = theta * jnp.dot(support, w_ref[...],
                              preferred_element_type=jnp.float32) \
            + (1.0 - theta) * support
        feat_ref[...] = p_ref[...] * jnp.maximum(out, 0.0)
    return body


def _layer_update(theta, parts, h0, q, p, W_init):
    return pl.pallas_call(
        _make_layer_body(theta),
        out_shape=jax.ShapeDtypeStruct((N, H), jnp.float32),
    )(parts, h0, q, p, W_init)


def _topk_body(scores_ref, rni_ref, sel_ref, scr):
    scr[...] = scores_ref[...]
    rows_i = lax.broadcasted_iota(jnp.int32, (NRP // 128, 128), 0)
    cols_i = lax.broadcasted_iota(jnp.int32, (NRP // 128, 128), 1)
    flat = rows_i * 128 + cols_i

    def tk_body(t, carry):
        sv = scr[...]
        m = jnp.max(sv)
        sel = jnp.min(jnp.where(sv == m, flat, jnp.int32(1 << 30)))
        sel_ref[t] = rni_ref[sel]
        scr[...] = jnp.where(flat == sel, -jnp.inf, sv)
        return carry

    lax.fori_loop(0, K, tk_body, 0)


def _topk(scores_pad, rni):
    return pl.pallas_call(
        _topk_body,
        out_shape=jax.ShapeDtypeStruct((K,), jnp.int32),
        in_specs=[
            pl.BlockSpec(memory_space=pltpu.VMEM),
            pl.BlockSpec(memory_space=pltpu.SMEM),
        ],
        out_specs=pl.BlockSpec(memory_space=pltpu.SMEM),
        scratch_shapes=[
            pltpu.VMEM((NRP // 128, 128), jnp.float32),
        ],
    )(scores_pad, rni)


def _make_pool_gru_body(theta):
    def body(parts_ref, h0_ref, q_ref, sel_ref, wi_ref, wih_ref, whh_ref,
             bih_ref, bhh_ref, out_ref, layer, xp, gi):
        hi = q_ref[...] * (parts_ref[0] + parts_ref[1])
        support = (1.0 - ALPHA) * hi + ALPHA * h0_ref[...]
        out = theta * jnp.dot(support, wi_ref[...],
                              preferred_element_type=jnp.float32) \
            + (1.0 - theta) * support
        layer[...] = jnp.maximum(out, 0.0)

        def gather_body(t, carry):
            nid = sel_ref[t]
            xp[pl.ds(t, 1), :] = layer[pl.ds(nid, 1), :]
            return carry

        lax.fori_loop(0, K, gather_body, 0)

        gi[...] = lax.dot_general(xp[...], wih_ref[...],
                                  (((1,), (1,)), ((), ())),
                                  preferred_element_type=jnp.float32) \
            + bih_ref[...]

        def gru_body(t, hv):
            gh = lax.dot_general(hv, whh_ref[...], (((1,), (1,)), ((), ())),
                                 preferred_element_type=jnp.float32) \
                + bhh_ref[...]
            git = gi[pl.ds(t, 1), :]
            r = jax.nn.sigmoid(git[:, 0:H] + gh[:, 0:H])
            z = jax.nn.sigmoid(git[:, H:2 * H] + gh[:, H:2 * H])
            n = jnp.tanh(git[:, 2 * H:3 * H] + r * gh[:, 2 * H:3 * H])
            hn = (1.0 - z) * n + z * hv
            out_ref[pl.ds(t, 1), :] = hn
            return hn

        lax.fori_loop(0, K, gru_body, jnp.zeros((1, H), jnp.float32))

    return body


def _pool_gru(theta, parts, h0, q, sel, W_init, W_ih, W_hh, b_ih, b_hh):
    return pl.pallas_call(
        _make_pool_gru_body(theta),
        out_shape=jax.ShapeDtypeStruct((K, H), jnp.float32),
        in_specs=[
            pl.BlockSpec(memory_space=pltpu.VMEM),
            pl.BlockSpec(memory_space=pltpu.VMEM),
            pl.BlockSpec(memory_space=pltpu.VMEM),
            pl.BlockSpec(memory_space=pltpu.SMEM),
            pl.BlockSpec(memory_space=pltpu.VMEM),
            pl.BlockSpec(memory_space=pltpu.VMEM),
            pl.BlockSpec(memory_space=pltpu.VMEM),
            pl.BlockSpec(memory_space=pltpu.VMEM),
            pl.BlockSpec(memory_space=pltpu.VMEM),
        ],
        scratch_shapes=[
            pltpu.VMEM((N, H), jnp.float32),
            pltpu.VMEM((K, H), jnp.float32),
            pltpu.VMEM((K, 3 * H), jnp.float32),
        ],
    )(parts, h0, q, sel, W_init, W_ih, W_hh, b_ih, b_hh)


def kernel(x, edge_index, remain_nodes_index, added_nodes_index, node_id,
           node_scores, W_hidden, b_hidden, Wsa0, bsa0, Wsa1, bsa1, a_vec,
           W_init, W_ih, W_hh, b_ih, b_hh):
    src = edge_index[0]
    dst = edge_index[1]
    src2 = src.reshape(E // CH2, CH2)
    dst2 = dst.reshape(E // CH2, CH2)
    src4 = src.reshape(E // CH4, CH4)
    dst4 = dst.reshape(E // CH4, CH4)

    h, al, ar, armax = _dense_pre(x, W_hidden, b_hidden, Wsa0, bsa0,
                                  Wsa1, bsa1, a_vec)
    armax16 = jnp.broadcast_to(armax.reshape(1), (16,))
    rs2p, emaxp = _edge_stats(src2, dst2, al.reshape(N), ar.reshape(N),
                              armax16)

    scores_pad = jnp.pad(node_scores, (0, NRP - NR),
                         constant_values=-jnp.inf).reshape(NRP // 128, 128)
    sel = _topk(scores_pad, remain_nodes_index)

    q, p, feat = _prep(rs2p.reshape(NCORES, N, 1), emaxp, al, ar, armax, h)

    theta1 = math.log(LAMDA / 1 + 1.0)
    parts = _spmm(feat, src4, dst4)
    feat = _layer_update(theta1, parts, h, q, p, W_init)

    theta2 = math.log(LAMDA / 2 + 1.0)
    parts = _spmm(feat, src4, dst4)
    return _pool_gru(theta2, parts, h, q, sel, W_init, W_ih, W_hh,
                     b_ih.reshape(1, 3 * H), b_hh.reshape(1, 3 * H))


# speedup vs baseline: 1.1792x; 1.0649x over previous
import math

import jax
import jax.numpy as jnp
from jax import lax
from jax.experimental import pallas as pl
from jax.experimental.pallas import tpu as pltpu
from jax.experimental.pallas import tpu_sc as plsc

N = 10000
E = 320000
D = 128
H = 128
NCONV = 2
LAMDA = 0.5
ALPHA = 0.1
NR = 9000
NRP = 9088
K = 128

NCORES = 2
NSUB = 16
NTILES = NCORES * NSUB
EPT = E // NTILES
CH2 = 80
CPT2 = EPT // CH2
CH4 = 40
NBUF = 5
CPT4 = EPT // CH4
RPS = N // NSUB

_vec_mesh = plsc.VectorSubcoreMesh(core_axis_name="c", subcore_axis_name="s")
_sc_params = pltpu.CompilerParams(use_tc_tiling_on_sc=False,
                                  needs_layout_passes=False)


def _dense_pre_body(x_ref, wh_ref, bh_ref, w0_ref, b0_ref, w1_ref, b1_ref,
                    av_ref, h_ref, al_ref, ar_ref, armax_ref, almax_ref):
    h = jnp.maximum(
        jnp.dot(x_ref[...], wh_ref[...], preferred_element_type=jnp.float32)
        + bh_ref[...], 0.0)
    h_ref[...] = h
    hl = jnp.dot(h, w0_ref[...], preferred_element_type=jnp.float32) + b0_ref[...]
    hr = jnp.dot(h, w1_ref[...], preferred_element_type=jnp.float32) + b1_ref[...]
    av = av_ref[...]

    def lrelu(v):
        return jnp.where(v > 0, v, 0.2 * v)

    al = jnp.sum(lrelu(hl) * av, axis=1, keepdims=True)
    ar = jnp.sum(lrelu(hr) * av, axis=1, keepdims=True)
    al_ref[...] = al
    ar_ref[...] = ar
    armax_ref[...] = jnp.reshape(jnp.max(ar), (1, 1))
    almax_ref[...] = jnp.reshape(jnp.max(al), (1, 1))


def _dense_pre(x, W_hidden, b_hidden, Wsa0, bsa0, Wsa1, bsa1, a_vec):
    return pl.pallas_call(
        _dense_pre_body,
        out_shape=(
            jax.ShapeDtypeStruct((N, H), jnp.float32),
            jax.ShapeDtypeStruct((N, 1), jnp.float32),
            jax.ShapeDtypeStruct((N, 1), jnp.float32),
            jax.ShapeDtypeStruct((1, 1), jnp.float32),
            jax.ShapeDtypeStruct((1, 1), jnp.float32),
        ),
    )(x, W_hidden, b_hidden.reshape(1, H), Wsa0, bsa0.reshape(1, H),
      Wsa1, bsa1.reshape(1, H), a_vec.reshape(1, H))


def _edge_stats_body(src_hbm, dst_hbm, ar_hbm, armax_hbm,
                     rs2_out,
                     artab, qtab, srcv, dstv, vals, armax_v, rs2_sh):
    c = lax.axis_index("c")
    s = lax.axis_index("s")
    wid = c * NSUB + s

    pltpu.sync_copy(ar_hbm, artab)
    pltpu.sync_copy(armax_hbm, armax_v)
    pltpu.sync_copy(src_hbm.at[pl.ds(wid * EPT, EPT)], srcv)
    pltpu.sync_copy(dst_hbm.at[pl.ds(wid * EPT, EPT)], dstv)

    @pl.when(s == 0)
    def _():
        @pl.loop(0, N, step=16)
        def _(i):
            qtab[pl.ds(i, 16)] = jnp.zeros((16,), jnp.float32)
        pltpu.sync_copy(qtab, rs2_sh)

    am = armax_v[...]

    @pl.loop(0, N, step=16)
    def _(i):
        qtab[pl.ds(i, 16)] = jnp.exp(artab[pl.ds(i, 16)] - am)

    plsc.subcore_barrier()

    @pl.loop(0, EPT, step=16)
    def _(j):
        vals[pl.ds(j, 16)] = plsc.load_gather(qtab, [dstv[pl.ds(j, 16)]])

    pltpu.sync_copy(vals, rs2_sh.at[srcv], add=True)

    plsc.subcore_barrier()

    @pl.when(s == 0)
    def _():
        pltpu.sync_copy(rs2_sh, rs2_out.at[c])


def _edge_stats(src, dst, ar, armax16):
    kfn = pl.kernel(
        _edge_stats_body,
        out_type=jax.ShapeDtypeStruct((NCORES, N), jnp.float32),
        mesh=_vec_mesh,
        scratch_types=[
            pltpu.VMEM((N,), jnp.float32),
            pltpu.VMEM((N,), jnp.float32),
            pltpu.VMEM((EPT,), jnp.int32),
            pltpu.VMEM((EPT,), jnp.int32),
            pltpu.VMEM((EPT,), jnp.float32),
            pltpu.VMEM((16,), jnp.float32),
            pltpu.VMEM_SHARED((N,), jnp.float32),
        ],
        compiler_params=_sc_params,
    )
    return kfn(src, dst, ar, armax16)


def _prep_body(rs2p_ref, al_ref, ar_ref, armax_ref, almax_ref, h_ref,
               q_ref, p_ref, feat_ref):
    armax = armax_ref[...]
    almax = almax_ref[...]
    rs2 = rs2p_ref[0] + rs2p_ref[1]
    t = jnp.exp(al_ref[...] - almax)
    p = t / (t * rs2 + 1e-16)
    q = jnp.exp(ar_ref[...] - armax)
    q_ref[...] = q
    p_ref[...] = p
    feat_ref[...] = p * h_ref[...]


def _prep(rs2p, al, ar, armax, almax, h):
    return pl.pallas_call(
        _prep_body,
        out_shape=(
            jax.ShapeDtypeStruct((N, 1), jnp.float32),
            jax.ShapeDtypeStruct((N, 1), jnp.float32),
            jax.ShapeDtypeStruct((N, H), jnp.float32),
        ),
    )(rs2p, al, ar, armax, almax, h)


def _spmm_body(feat_hbm, src_hbm, dst_hbm, out_hbm, sidx, didx,
               rows0, rows1, rows2, rows3, rows4, acc,
               sem0, sem1, sem2, sem3, sem4):
    c = lax.axis_index("c")
    s = lax.axis_index("s")
    wid = c * NSUB + s
    bufs = (rows0, rows1, rows2, rows3, rows4)
    sems = (sem0, sem1, sem2, sem3, sem4)

    pltpu.sync_copy(src_hbm.at[pl.ds(wid * CPT4, CPT4)], sidx)
    pltpu.sync_copy(dst_hbm.at[pl.ds(wid * CPT4, CPT4)], didx)

    @pl.loop(0, CH4)
    def _(r):
        @pl.loop(0, D, step=16)
        def _(k):
            rows0[r, pl.ds(k, 16)] = jnp.zeros((16,), jnp.float32)

    @pl.loop(0, RPS // CH4)
    def _(j):
        pltpu.sync_copy(rows0, acc.at[pl.ds(s * RPS + j * CH4, CH4)])

    pltpu.sync_copy(rows0, acc.at[pl.ds(s * RPS + RPS - CH4, CH4)])

    plsc.subcore_barrier()

    for b in range(NBUF):
        pltpu.async_copy(feat_hbm.at[sidx.at[b]], bufs[b], sems[b])

    @pl.loop(0, CPT4, step=NBUF)
    def _(ch):
        for b in range(NBUF):
            pltpu.make_async_copy(feat_hbm.at[sidx.at[ch + b]],
                                  bufs[b], sems[b]).wait()
            pltpu.sync_copy(bufs[b], acc.at[didx.at[ch + b]], add=True)

            @pl.when(ch + b + NBUF < CPT4)
            def _(b=b):
                pltpu.async_copy(feat_hbm.at[sidx.at[ch + b + NBUF]],
                                 bufs[b], sems[b])

    plsc.subcore_barrier()

    pltpu.sync_copy(acc.at[pl.ds(s * RPS, RPS)],
                    out_hbm.at[c, pl.ds(s * RPS, RPS)])


def _spmm(feat, src4, dst4):
    kfn = pl.kernel(
        _spmm_body,
        out_type=jax.ShapeDtypeStruct((NCORES, N, D), jnp.float32),
        mesh=_vec_mesh,
        scratch_types=[
            pltpu.VMEM((CPT4, CH4), jnp.int32),
            pltpu.VMEM((CPT4, CH4), jnp.int32),
            pltpu.VMEM((CH4, D), jnp.float32),
            pltpu.VMEM((CH4, D), jnp.float32),
            pltpu.VMEM((CH4, D), jnp.float32),
            pltpu.VMEM((CH4, D), jnp.float32),
            pltpu.VMEM((CH4, D), jnp.float32),
            pltpu.VMEM_SHARED((N, D), jnp.float32),
            pltpu.SemaphoreType.DMA,
            pltpu.SemaphoreType.DMA,
            pltpu.SemaphoreType.DMA,
            pltpu.SemaphoreType.DMA,
            pltpu.SemaphoreType.DMA,
        ],
        compiler_params=_sc_params,
    )
    return kfn(feat, src4, dst4)


def _make_layer_body(theta):
    def body(parts_ref, h0_ref, q_ref, p_ref, w_ref, feat_ref):
        hi = q_ref[...] * (parts_ref[0] + parts_ref[1])
        support = (1.0 - ALPHA) * hi + ALPHA * h0_ref[...]
        out = theta * jnp.dot(support, w_ref[...],
                              preferred_element_type=jnp.float32) \
            + (1.0 - theta) * support
        feat_ref[...] = p_ref[...] * jnp.maximum(out, 0.0)
    return body


def _layer_update(theta, parts, h0, q, p, W_init):
    return pl.pallas_call(
        _make_layer_body(theta),
        out_shape=jax.ShapeDtypeStruct((N, H), jnp.float32),
    )(parts, h0, q, p, W_init)


def _topk_body(scores_ref, rni_ref, sel_ref, scr):
    scr[...] = scores_ref[...]
    rows_i = lax.broadcasted_iota(jnp.int32, (NRP // 128, 128), 0)
    cols_i = lax.broadcasted_iota(jnp.int32, (NRP // 128, 128), 1)
    flat = rows_i * 128 + cols_i

    def tk_body(t, carry):
        sv = scr[...]
        m = jnp.max(sv)
        sel = jnp.min(jnp.where(sv == m, flat, jnp.int32(1 << 30)))
        sel_ref[t] = rni_ref[sel]
        scr[...] = jnp.where(flat == sel, -jnp.inf, sv)
        return carry

    lax.fori_loop(0, K, tk_body, 0)


def _topk(scores_pad, rni):
    return pl.pallas_call(
        _topk_body,
        out_shape=jax.ShapeDtypeStruct((K,), jnp.int32),
        in_specs=[
            pl.BlockSpec(memory_space=pltpu.VMEM),
            pl.BlockSpec(memory_space=pltpu.SMEM),
        ],
        out_specs=pl.BlockSpec(memory_space=pltpu.SMEM),
        scratch_shapes=[
            pltpu.VMEM((NRP // 128, 128), jnp.float32),
        ],
    )(scores_pad, rni)


def _make_pool_gru_body(theta):
    def body(parts_ref, h0_ref, q_ref, sel_ref, wi_ref, wih_ref, whh_ref,
             bih_ref, bhh_ref, out_ref, layer, xp, gi):
        hi = q_ref[...] * (parts_ref[0] + parts_ref[1])
        support = (1.0 - ALPHA) * hi + ALPHA * h0_ref[...]
        out = theta * jnp.dot(support, wi_ref[...],
                              preferred_element_type=jnp.float32) \
            + (1.0 - theta) * support
        layer[...] = jnp.maximum(out, 0.0)

        def gather_body(t, carry):
            nid = sel_ref[t]
            xp[pl.ds(t, 1), :] = layer[pl.ds(nid, 1), :]
            return carry

        lax.fori_loop(0, K, gather_body, 0)

        gi[...] = lax.dot_general(xp[...], wih_ref[...],
                                  (((1,), (1,)), ((), ())),
                                  preferred_element_type=jnp.float32) \
            + bih_ref[...]

        def gru_body(t, hv):
            gh = lax.dot_general(hv, whh_ref[...], (((1,), (1,)), ((), ())),
                                 preferred_element_type=jnp.float32) \
                + bhh_ref[...]
            git = gi[pl.ds(t, 1), :]
            r = jax.nn.sigmoid(git[:, 0:H] + gh[:, 0:H])
            z = jax.nn.sigmoid(git[:, H:2 * H] + gh[:, H:2 * H])
            n = jnp.tanh(git[:, 2 * H:3 * H] + r * gh[:, 2 * H:3 * H])
            hn = (1.0 - z) * n + z * hv
            out_ref[pl.ds(t, 1), :] = hn
            return hn

        lax.fori_loop(0, K, gru_body, jnp.zeros((1, H), jnp.float32))

    return body


def _pool_gru(theta, parts, h0, q, sel, W_init, W_ih, W_hh, b_ih, b_hh):
    return pl.pallas_call(
        _make_pool_gru_body(theta),
        out_shape=jax.ShapeDtypeStruct((K, H), jnp.float32),
        in_specs=[
            pl.BlockSpec(memory_space=pltpu.VMEM),
            pl.BlockSpec(memory_space=pltpu.VMEM),
            pl.BlockSpec(memory_space=pltpu.VMEM),
            pl.BlockSpec(memory_space=pltpu.SMEM),
            pl.BlockSpec(memory_space=pltpu.VMEM),
            pl.BlockSpec(memory_space=pltpu.VMEM),
            pl.BlockSpec(memory_space=pltpu.VMEM),
            pl.BlockSpec(memory_space=pltpu.VMEM),
            pl.BlockSpec(memory_space=pltpu.VMEM),
        ],
        scratch_shapes=[
            pltpu.VMEM((N, H), jnp.float32),
            pltpu.VMEM((K, H), jnp.float32),
            pltpu.VMEM((K, 3 * H), jnp.float32),
        ],
    )(parts, h0, q, sel, W_init, W_ih, W_hh, b_ih, b_hh)


def kernel(x, edge_index, remain_nodes_index, added_nodes_index, node_id,
           node_scores, W_hidden, b_hidden, Wsa0, bsa0, Wsa1, bsa1, a_vec,
           W_init, W_ih, W_hh, b_ih, b_hh):
    src = edge_index[0]
    dst = edge_index[1]
    src4 = src.reshape(E // CH4, CH4)
    dst4 = dst.reshape(E // CH4, CH4)

    h, al, ar, armax, almax = _dense_pre(x, W_hidden, b_hidden, Wsa0, bsa0,
                                         Wsa1, bsa1, a_vec)
    armax16 = jnp.broadcast_to(armax.reshape(1), (16,))
    rs2p = _edge_stats(src, dst, ar.reshape(N), armax16)

    scores_pad = jnp.pad(node_scores, (0, NRP - NR),
                         constant_values=-jnp.inf).reshape(NRP // 128, 128)
    sel = _topk(scores_pad, remain_nodes_index)

    q, p, feat = _prep(rs2p.reshape(NCORES, N, 1), al, ar, armax, almax, h)

    theta1 = math.log(LAMDA / 1 + 1.0)
    parts = _spmm(feat, src4, dst4)
    feat = _layer_update(theta1, parts, h, q, p, W_init)

    theta2 = math.log(LAMDA / 2 + 1.0)
    parts = _spmm(feat, src4, dst4)
    return _pool_gru(theta2, parts, h, q, sel, W_init, W_ih, W_hh,
                     b_ih.reshape(1, 3 * H), b_hh.reshape(1, 3 * H))
